# baseline (speedup 1.0000x reference)
import functools

import jax
import jax.numpy as jnp
from jax import lax
from jax.experimental import pallas as pl
from jax.experimental.pallas import tpu as pltpu
from jax.experimental.pallas import tpu_sc as plsc

_SEQ = 8192
_DIM = 1024
_NUM_CORES = 2
_NUM_SUBCORES = 16
_NW = _NUM_CORES * _NUM_SUBCORES
_ROWS = _SEQ // _NW


_mesh = plsc.VectorSubcoreMesh(core_axis_name="c", subcore_axis_name="s")


@functools.partial(
    pl.kernel,
    mesh=_mesh,
    out_type=jax.ShapeDtypeStruct((_SEQ, _DIM), jnp.float32),
)
def _pos_copy(emb_hbm, out_hbm):
    wid = lax.axis_index("s") * _NUM_CORES + lax.axis_index("c")
    base = wid * _ROWS
    pltpu.sync_copy(emb_hbm.at[pl.ds(base, _ROWS)], out_hbm.at[pl.ds(base, _ROWS)])


def kernel(x, emb):
    del x
    return _pos_copy(emb)[None]


# speedup vs baseline: 24.3336x; 24.3336x over previous
import functools

import jax
import jax.numpy as jnp
from jax import lax
from jax.experimental import pallas as pl
from jax.experimental.pallas import tpu as pltpu
from jax.experimental.pallas import tpu_sc as plsc

_SEQ = 8192
_DIM = 1024
_NUM_CORES = 2
_NUM_SUBCORES = 16
_NW = _NUM_CORES * _NUM_SUBCORES
_ROWS = _SEQ // _NW
_CHUNK = 32
_T = _ROWS // _CHUNK


_mesh = plsc.VectorSubcoreMesh(core_axis_name="c", subcore_axis_name="s")


@functools.partial(
    pl.kernel,
    mesh=_mesh,
    out_type=jax.ShapeDtypeStruct((_SEQ, _DIM), jnp.float32),
    scratch_types=[
        pltpu.VMEM((2, _CHUNK, _DIM), jnp.float32),
        pltpu.SemaphoreType.DMA,
        pltpu.SemaphoreType.DMA,
        pltpu.SemaphoreType.DMA,
        pltpu.SemaphoreType.DMA,
    ],
)
def _pos_copy(emb_hbm, out_hbm, buf, s_in0, s_in1, s_out0, s_out1):
    s_in = (s_in0, s_in1)
    s_out = (s_out0, s_out1)
    wid = lax.axis_index("s") * _NUM_CORES + lax.axis_index("c")
    base = wid * _ROWS

    in_h = [None] * _T
    out_h = [None] * _T
    in_h[0] = pltpu.async_copy(
        emb_hbm.at[pl.ds(base, _CHUNK)], buf.at[0], s_in[0])
    for t in range(_T):
        b = t % 2
        if t + 1 < _T:
            if t >= 1:
                out_h[t - 1].wait()
            in_h[t + 1] = pltpu.async_copy(
                emb_hbm.at[pl.ds(base + (t + 1) * _CHUNK, _CHUNK)],
                buf.at[(t + 1) % 2], s_in[(t + 1) % 2])
        in_h[t].wait()
        out_h[t] = pltpu.async_copy(
            buf.at[b], out_hbm.at[pl.ds(base + t * _CHUNK, _CHUNK)], s_out[b])
    out_h[_T - 2].wait()
    out_h[_T - 1].wait()


def kernel(x, emb):
    del x
    return _pos_copy(emb)[None]
